# baseline (speedup 1.0000x reference)
import jax
import jax.numpy as jnp
from jax import lax
from jax.experimental import pallas as pl
from jax.experimental.pallas import tpu as pltpu
from jax.experimental.pallas import tpu_sc as plsc

_B = 16384
_D = 128
_L = 16
_NCORES = 2
_NSUB = 16
_NW = _NCORES * _NSUB
_RPW = _B // _NW
_CH = 64
_NCHUNK = _RPW // _CH
_NBUF = 3
_NG = _CH // _L


def _body(x_hbm, y_hbm, fcw_hbm, fcb_hbm, emb_hbm, out_hbm,
          idx_v, x_v, e_v, out_v, par_v, fcb_v, tr_v,
          si, sx0, sx1, sx2, se0, se1, se2, so):
    cid = lax.axis_index("c")
    sid = lax.axis_index("s")
    base = (sid * _NCORES + cid) * _RPW

    hidx = pltpu.async_copy(y_hbm.at[pl.ds(pl.multiple_of(base, _CH), _RPW)],
                            idx_v, si)
    pltpu.sync_copy(fcw_hbm, par_v)
    pltpu.sync_copy(fcb_hbm, fcb_v)
    fcw = [par_v[0, pl.ds(_L * j, _L)] for j in range(_D // _L)]
    fcb_vec = plsc.load_gather(fcb_v, [jnp.zeros((_L,), jnp.int32)])
    gbase = lax.iota(jnp.int32, _L) * _L
    hidx.wait()

    semx = [sx0, sx1, sx2]
    seme = [se0, se1, se2]

    def issue(c):
        b = c % _NBUF
        r0 = pl.multiple_of(base + c * _CH, _CH)
        hx = pltpu.async_copy(x_hbm.at[pl.ds(r0, _CH)], x_v.at[b], semx[b])
        he = pltpu.async_copy(
            emb_hbm.at[idx_v.at[pl.ds(c * _CH, _CH)]], e_v.at[b], seme[b])
        return hx, he

    pend = [issue(c) for c in range(_NBUF - 1)]
    out_handles = []
    for c in range(_NCHUNK):
        b = c % _NBUF
        hx, he = pend.pop(0)
        if c + _NBUF - 1 < _NCHUNK:
            pend.append(issue(c + _NBUF - 1))
        hx.wait()
        he.wait()

        @plsc.parallel_loop(0, _NG, step=1, unroll=2)
        def group(g, b=b, c=c):
            tbase = pl.multiple_of(g * _L * _L, _L)
            for rr in range(_L):
                r = g * _L + rr
                acc = None
                for j in range(_D // _L):
                    xv = x_v[b, r, pl.ds(_L * j, _L)]
                    ev = e_v[b, r, pl.ds(_L * j, _L)]
                    t = xv * (ev + fcw[j])
                    acc = t if acc is None else acc + t
                tr_v[pl.ds(tbase + rr * _L, _L)] = acc
            outvec = fcb_vec
            for col in range(_L):
                outvec = outvec + plsc.load_gather(tr_v, [tbase + gbase + col])
            out_v[c, pl.ds(pl.multiple_of(g * _L, _L), _L)] = outvec

        r0 = pl.multiple_of(base + c * _CH, _CH)
        out_handles.append(
            pltpu.async_copy(out_v.at[c], out_hbm.at[pl.ds(r0, _CH)], so))
    for h in out_handles:
        h.wait()


_sc_call = pl.kernel(
    _body,
    out_type=jax.ShapeDtypeStruct((_B,), jnp.float32),
    mesh=plsc.VectorSubcoreMesh(
        core_axis_name="c", subcore_axis_name="s",
        num_cores=_NCORES, num_subcores=_NSUB),
    compiler_params=pltpu.CompilerParams(needs_layout_passes=False),
    scratch_types=[
        pltpu.VMEM((_RPW,), jnp.int32),
        pltpu.VMEM((_NBUF, _CH, _D), jnp.float32),
        pltpu.VMEM((_NBUF, _CH, _D), jnp.float32),
        pltpu.VMEM((_NCHUNK, _CH), jnp.float32),
        pltpu.VMEM((1, _D), jnp.float32),
        pltpu.VMEM((1,), jnp.float32),
        pltpu.VMEM((_NG * _L * _L,), jnp.float32),
        pltpu.SemaphoreType.DMA,
        pltpu.SemaphoreType.DMA,
        pltpu.SemaphoreType.DMA,
        pltpu.SemaphoreType.DMA,
        pltpu.SemaphoreType.DMA,
        pltpu.SemaphoreType.DMA,
        pltpu.SemaphoreType.DMA,
        pltpu.SemaphoreType.DMA,
    ],
)


def kernel(x, y, fc_w, fc_b, emb):
    return _sc_call(x, y.astype(jnp.int32), fc_w, fc_b, emb)


# speedup vs baseline: 1.1125x; 1.1125x over previous
import jax
import jax.numpy as jnp
from jax import lax
from jax.experimental import pallas as pl
from jax.experimental.pallas import tpu as pltpu
from jax.experimental.pallas import tpu_sc as plsc

_B = 16384
_D = 128
_L = 16
_NCORES = 2
_NSUB = 16
_NW = _NCORES * _NSUB
_RPW = _B // _NW
_CH = 128
_NCHUNK = _RPW // _CH
_NBUF = 2
_NG = _CH // _L


def _body(x_hbm, y_hbm, fcw_hbm, fcb_hbm, emb_hbm, out_hbm,
          idx_v, x_v, e_v, out_v, par_v, fcb_v, tr_v,
          si, sx0, sx1, se0, se1, so):
    cid = lax.axis_index("c")
    sid = lax.axis_index("s")
    base = (sid * _NCORES + cid) * _RPW

    hidx = pltpu.async_copy(y_hbm.at[pl.ds(pl.multiple_of(base, _CH), _RPW)],
                            idx_v, si)
    pltpu.sync_copy(fcw_hbm, par_v)
    pltpu.sync_copy(fcb_hbm, fcb_v)
    fcw = [par_v[0, pl.ds(_L * j, _L)] for j in range(_D // _L)]
    fcb_vec = plsc.load_gather(fcb_v, [jnp.zeros((_L,), jnp.int32)])
    gbase = lax.iota(jnp.int32, _L) * _L
    hidx.wait()

    semx = [sx0, sx1]
    seme = [se0, se1]

    def issue(c):
        b = c % _NBUF
        r0 = pl.multiple_of(base + c * _CH, _CH)
        hx = pltpu.async_copy(x_hbm.at[pl.ds(r0, _CH)], x_v.at[b], semx[b])
        he = pltpu.async_copy(
            emb_hbm.at[idx_v.at[pl.ds(c * _CH, _CH)]], e_v.at[b], seme[b])
        return hx, he

    pend = [issue(c) for c in range(_NBUF - 1)]
    out_handles = []
    for c in range(_NCHUNK):
        b = c % _NBUF
        hx, he = pend.pop(0)
        if c + _NBUF - 1 < _NCHUNK:
            pend.append(issue(c + _NBUF - 1))
        hx.wait()
        he.wait()

        @plsc.parallel_loop(0, _NG, step=1, unroll=2)
        def group(g, b=b, c=c):
            tbase = pl.multiple_of(g * _L * _L, _L)
            for rr in range(_L):
                r = g * _L + rr
                acc = None
                for j in range(_D // _L):
                    xv = x_v[b, r, pl.ds(_L * j, _L)]
                    ev = e_v[b, r, pl.ds(_L * j, _L)]
                    t = xv * (ev + fcw[j])
                    acc = t if acc is None else acc + t
                tr_v[pl.ds(tbase + rr * _L, _L)] = acc
            outvec = fcb_vec
            for col in range(_L):
                outvec = outvec + plsc.load_gather(tr_v, [tbase + gbase + col])
            out_v[c, pl.ds(pl.multiple_of(g * _L, _L), _L)] = outvec

        r0 = pl.multiple_of(base + c * _CH, _CH)
        out_handles.append(
            pltpu.async_copy(out_v.at[c], out_hbm.at[pl.ds(r0, _CH)], so))
    for h in out_handles:
        h.wait()


_sc_call = pl.kernel(
    _body,
    out_type=jax.ShapeDtypeStruct((_B,), jnp.float32),
    mesh=plsc.VectorSubcoreMesh(
        core_axis_name="c", subcore_axis_name="s",
        num_cores=_NCORES, num_subcores=_NSUB),
    compiler_params=pltpu.CompilerParams(needs_layout_passes=False),
    scratch_types=[
        pltpu.VMEM((_RPW,), jnp.int32),
        pltpu.VMEM((_NBUF, _CH, _D), jnp.float32),
        pltpu.VMEM((_NBUF, _CH, _D), jnp.float32),
        pltpu.VMEM((_NCHUNK, _CH), jnp.float32),
        pltpu.VMEM((1, _D), jnp.float32),
        pltpu.VMEM((1,), jnp.float32),
        pltpu.VMEM((_NG * _L * _L,), jnp.float32),
        pltpu.SemaphoreType.DMA,
        pltpu.SemaphoreType.DMA,
        pltpu.SemaphoreType.DMA,
        pltpu.SemaphoreType.DMA,
        pltpu.SemaphoreType.DMA,
        pltpu.SemaphoreType.DMA,
    ],
)


def kernel(x, y, fc_w, fc_b, emb):
    return _sc_call(x, y.astype(jnp.int32), fc_w, fc_b, emb)


# speedup vs baseline: 1.1377x; 1.0227x over previous
import jax
import jax.numpy as jnp
from jax import lax
from jax.experimental import pallas as pl
from jax.experimental.pallas import tpu as pltpu
from jax.experimental.pallas import tpu_sc as plsc

_B = 16384
_D = 128
_L = 16
_NCORES = 2
_NSUB = 16
_NW = _NCORES * _NSUB
_RPW = _B // _NW
_CH = 128
_NCHUNK = _RPW // _CH
_NG = _CH // _L


def _tree_sum(vs):
    while len(vs) > 1:
        vs = [a + b for a, b in zip(vs[::2], vs[1::2])]
    return vs[0]


def _body(x_hbm, y_hbm, fcw_hbm, fcb_hbm, emb_hbm, out_hbm,
          idx_v, x_v, e_v, out_v, par_v, fcb_v, tr_v,
          sx0, sx1, se0, se1, so):
    cid = lax.axis_index("c")
    sid = lax.axis_index("s")
    base = (sid * _NCORES + cid) * _RPW

    semx = [sx0, sx1]
    seme = [se0, se1]

    def issue(c):
        b = c % 2
        r0 = pl.multiple_of(base + c * _CH, _CH)
        pltpu.sync_copy(y_hbm.at[pl.ds(r0, _CH)], idx_v.at[b])
        hx = pltpu.async_copy(x_hbm.at[pl.ds(r0, _CH)], x_v.at[b], semx[b])
        he = pltpu.async_copy(emb_hbm.at[idx_v.at[b]], e_v.at[b], seme[b])
        return hx, he

    pend = issue(0)
    pltpu.sync_copy(fcw_hbm, par_v)
    pltpu.sync_copy(fcb_hbm, fcb_v)
    fcw = [par_v[0, pl.ds(_L * j, _L)] for j in range(_D // _L)]
    fcb_vec = plsc.load_gather(fcb_v, [jnp.zeros((_L,), jnp.int32)])
    gbase = lax.iota(jnp.int32, _L) * _L

    out_handles = []
    for c in range(_NCHUNK):
        b = c % 2
        hx, he = pend
        if c + 1 < _NCHUNK:
            pend = issue(c + 1)
        hx.wait()
        he.wait()

        @plsc.parallel_loop(0, _NG, step=1, unroll=2)
        def group(g, b=b, c=c):
            tbase = pl.multiple_of(g * _L * _L, _L)
            for rr in range(_L):
                r = g * _L + rr
                parts = []
                for j in range(_D // _L):
                    xv = x_v[b, r, pl.ds(_L * j, _L)]
                    ev = e_v[b, r, pl.ds(_L * j, _L)]
                    parts.append(xv * (ev + fcw[j]))
                tr_v[pl.ds(tbase + rr * _L, _L)] = _tree_sum(parts)
            cols = [plsc.load_gather(tr_v, [tbase + gbase + col])
                    for col in range(_L)]
            out_v[c, pl.ds(pl.multiple_of(g * _L, _L), _L)] = (
                _tree_sum(cols) + fcb_vec)

        r0 = pl.multiple_of(base + c * _CH, _CH)
        out_handles.append(
            pltpu.async_copy(out_v.at[c], out_hbm.at[pl.ds(r0, _CH)], so))
    for h in out_handles:
        h.wait()


_sc_call = pl.kernel(
    _body,
    out_type=jax.ShapeDtypeStruct((_B,), jnp.float32),
    mesh=plsc.VectorSubcoreMesh(
        core_axis_name="c", subcore_axis_name="s",
        num_cores=_NCORES, num_subcores=_NSUB),
    compiler_params=pltpu.CompilerParams(needs_layout_passes=False),
    scratch_types=[
        pltpu.VMEM((2, _CH), jnp.int32),
        pltpu.VMEM((2, _CH, _D), jnp.float32),
        pltpu.VMEM((2, _CH, _D), jnp.float32),
        pltpu.VMEM((_NCHUNK, _CH), jnp.float32),
        pltpu.VMEM((1, _D), jnp.float32),
        pltpu.VMEM((1,), jnp.float32),
        pltpu.VMEM((_NG * _L * _L,), jnp.float32),
        pltpu.SemaphoreType.DMA,
        pltpu.SemaphoreType.DMA,
        pltpu.SemaphoreType.DMA,
        pltpu.SemaphoreType.DMA,
        pltpu.SemaphoreType.DMA,
    ],
)


def kernel(x, y, fc_w, fc_b, emb):
    return _sc_call(x, y.astype(jnp.int32), fc_w, fc_b, emb)
